# Initial kernel scaffold; baseline (speedup 1.0000x reference)
import jax
import jax.numpy as jnp
from jax.experimental import pallas as pl


def kernel(h, edge_index, e, W_emb, b_emb, Wmlp, bmlp, gamma, beta, Wn, bn_, Wg, bg, W_ih, W_hh, b_ih, b_hh):
    raise NotImplementedError("write your pallas kernel here")


# speedup vs baseline: 1.0136x; 1.0136x over previous
import jax
import jax.numpy as jnp
from jax.experimental import pallas as pl

L = 4


def _emb_body(h_ref, w_ref, b_ref, o_ref):
    h = h_ref[...]
    w = w_ref[...]
    o_ref[...] = h[:, 0:1] * w[0:1, :] + h[:, 1:2] * w[1:2, :] + b_ref[...]


def kernel(h, edge_index, e, W_emb, b_emb, Wmlp, bmlp, gamma, beta, Wn, bn_, Wg, bg, W_ih, W_hh, b_ih, b_hh):
    n = h.shape[0]
    H = W_emb.shape[1]
    src = edge_index[0]
    dst = edge_index[1]

    x = pl.pallas_call(
        _emb_body,
        out_shape=jax.ShapeDtypeStruct((n, H), jnp.float32),
    )(h, W_emb, b_emb.reshape(1, H))

    hidden = [x]
    for l in range(L):
        neigh = jax.ops.segment_sum(x[src], dst, num_segments=n)
        t = (x + neigh) @ Wmlp[l] + bmlp[l]
        mu = jnp.mean(t, axis=0)
        var = jnp.var(t, axis=0)
        t = (t - mu) / jnp.sqrt(var + 1e-5) * gamma[l] + beta[l]
        x = x + jax.nn.relu(t)
        hidden.append(x)

    gates = (b_ih + b_hh).reshape(1, 4 * H)
    i_g, f_g, g_g, o_g = jnp.split(gates, 4, axis=1)
    c = jax.nn.sigmoid(i_g) * jnp.tanh(g_g)
    q = jax.nn.sigmoid(o_g) * jnp.tanh(c)

    score_nodes = jnp.zeros((n, 3), dtype=x.dtype)
    score_graph = jnp.zeros((1, 3), dtype=x.dtype)
    for i in range(L + 1):
        hr = hidden[i]
        score_nodes = score_nodes + hr @ Wn[i] + bn_[i]
        att = hr @ q.T
        alpha = jax.nn.softmax(att, axis=0)
        readout = alpha.T @ hr
        pooled = jnp.concatenate([q, readout], axis=1)
        score_graph = score_graph + pooled @ Wg[i] + bg[i]
    return (score_nodes, score_graph)


# speedup vs baseline: 3.0241x; 2.9837x over previous
import functools

import jax
import jax.numpy as jnp
from jax import lax
from jax.experimental import pallas as pl
from jax.experimental.pallas import tpu as pltpu
from jax.experimental.pallas import tpu_sc as plsc

F32 = jnp.float32


def _bf(v):
    return v.astype(jnp.bfloat16)


def _dot(a, b):
    return jnp.dot(_bf(a), _bf(b), preferred_element_type=F32)


def _make_segsum(n, hh, e):
    ns = 16
    ept = e // ns
    ch = 80
    nch = ept // ch
    own = (n // ns) & ~7
    tail = n - own * ns
    zsweep = -(-(own + tail) // ch)
    nfull = own // ch
    rem = own - nfull * ch
    assert ept % ch == 0 and e % ns == 0 and ch % 8 == 0
    assert rem % 8 == 0 and tail % 8 == 0 and own % 8 == 0

    mesh = plsc.VectorSubcoreMesh(core_axis_name="c", subcore_axis_name="s")

    @functools.partial(
        pl.kernel,
        mesh=mesh,
        out_type=jax.ShapeDtypeStruct((2 * n, hh), F32),
        scratch_types=[
            pltpu.VMEM((ch,), jnp.int32),
            pltpu.VMEM((ch,), jnp.int32),
            pltpu.VMEM((ch, hh), F32),
            pltpu.VMEM_SHARED((n, hh), F32),
            pltpu.SemaphoreType.DMA,
        ],
    )
    def segsum(x_hbm, src2_hbm, dst_hbm, out_hbm, sidx, didx, rows, acc, sem):
        c = lax.axis_index("c")
        s = lax.axis_index("s")

        def zrow(i, carry):
            def zcol(j, carry2):
                rows[i, pl.ds(j * 16, 16)] = jnp.zeros((16,), F32)
                return carry2
            return lax.fori_loop(0, hh // 16, zcol, carry)
        lax.fori_loop(0, ch, zrow, 0)

        r0 = s * own
        for m in range(zsweep - 1):
            pltpu.sync_copy(rows, acc.at[pl.ds(r0 + m * ch, ch)])
        lastz = own + tail - (zsweep - 1) * ch
        pltpu.sync_copy(rows.at[pl.ds(0, lastz)],
                        acc.at[pl.ds(r0 + (zsweep - 1) * ch, lastz)])
        plsc.subcore_barrier()

        ebase_s = c * e + s * ept
        ebase_d = s * ept

        def step(k, carry):
            off = k * ch
            pltpu.sync_copy(src2_hbm.at[pl.ds(ebase_s + off, ch)], sidx)
            pltpu.async_copy(x_hbm.at[sidx], rows, sem).wait()
            pltpu.sync_copy(dst_hbm.at[pl.ds(ebase_d + off, ch)], didx)
            pltpu.sync_copy(rows, acc.at[didx], add=True)
            return carry
        lax.fori_loop(0, nch, step, 0)

        plsc.subcore_barrier()
        for m in range(nfull):
            pltpu.sync_copy(acc.at[pl.ds(r0 + m * ch, ch)],
                            out_hbm.at[pl.ds(c * n + r0 + m * ch, ch)])
        if rem:
            pltpu.sync_copy(acc.at[pl.ds(r0 + nfull * ch, rem)],
                            out_hbm.at[pl.ds(c * n + r0 + nfull * ch, rem)])

        @pl.when(s == ns - 1)
        def _():
            if tail:
                pltpu.sync_copy(acc.at[pl.ds(ns * own, tail)],
                                out_hbm.at[pl.ds(c * n + ns * own, tail)])

    return segsum


def _embed_body(h_ref, w_ref, b_ref, o_ref):
    hb = _bf(h_ref[...]).astype(F32)
    wb = _bf(w_ref[...]).astype(F32)
    x = hb[:, 0:1] * wb[0:1, :] + hb[:, 1:2] * wb[1:2, :] + b_ref[...]
    o_ref[0] = x[:, :128]
    o_ref[1] = x[:, 128:]


def _la_body(nb, x_ref, ng_ref, w_ref, b_ref, t_ref, st_ref, sums):
    i = pl.program_id(0)
    z0 = x_ref[0] + ng_ref[0]
    z1 = x_ref[1] + ng_ref[1]
    z0b, z1b = _bf(z0), _bf(z1)
    w = w_ref[...]
    t0 = (jnp.dot(z0b, _bf(w[:128, :128]), preferred_element_type=F32)
          + jnp.dot(z1b, _bf(w[128:, :128]), preferred_element_type=F32)
          + b_ref[0])
    t1 = (jnp.dot(z0b, _bf(w[:128, 128:]), preferred_element_type=F32)
          + jnp.dot(z1b, _bf(w[128:, 128:]), preferred_element_type=F32)
          + b_ref[1])
    t_ref[0] = t0
    t_ref[1] = t1

    @pl.when(i == 0)
    def _():
        sums[...] = jnp.zeros_like(sums)

    sums[0, :] += jnp.sum(t0, axis=0)
    sums[1, :] += jnp.sum(t1, axis=0)
    sums[2, :] += jnp.sum(t0 * t0, axis=0)
    sums[3, :] += jnp.sum(t1 * t1, axis=0)

    @pl.when(i == nb - 1)
    def _():
        st_ref[...] = sums[...]


def _lb_body(n, x_ref, t_ref, st_ref, g_ref, be_ref, o_ref):
    st = st_ref[...]
    mu0 = st[0:1] / n
    mu1 = st[1:2] / n
    var0 = st[2:3] / n - mu0 * mu0
    var1 = st[3:4] / n - mu1 * mu1
    t0 = (t_ref[0] - mu0) / jnp.sqrt(var0 + 1e-5) * g_ref[0:1] + be_ref[0:1]
    t1 = (t_ref[1] - mu1) / jnp.sqrt(var1 + 1e-5) * g_ref[1:2] + be_ref[1:2]
    o_ref[0] = x_ref[0] + jnp.maximum(t0, 0.0)
    o_ref[1] = x_ref[1] + jnp.maximum(t1, 0.0)


def _q_vec(bih_ref, bhh_ref):
    g = bih_ref[...] + bhh_ref[...]
    i_g, f_g, g_g, o_g = g[0:2], g[2:4], g[4:6], g[6:8]
    del f_g
    c = jax.nn.sigmoid(i_g) * jnp.tanh(g_g)
    return jax.nn.sigmoid(o_g) * jnp.tanh(c)


def _ra_body(h0, h1, h2, h3, h4, wn_ref, bn_ref, bih_ref, bhh_ref, x_ref):
    q = _q_vec(bih_ref, bhh_ref)
    hrs = (h0, h1, h2, h3, h4)
    sn = jnp.sum(bn_ref[...], axis=0, keepdims=True)
    sn = jnp.broadcast_to(sn, (h0.shape[1], 3))
    atts = []
    for t in range(5):
        hr = hrs[t]
        sn = (sn + jnp.dot(_bf(hr[0]), _bf(wn_ref[t, 0]),
                           preferred_element_type=F32)
              + jnp.dot(_bf(hr[1]), _bf(wn_ref[t, 1]),
                        preferred_element_type=F32))
        att = (jnp.sum(hr[0] * q[0:1], axis=1, keepdims=True)
               + jnp.sum(hr[1] * q[1:2], axis=1, keepdims=True))
        atts.append(att)
    x_ref[...] = jnp.concatenate([sn] + atts, axis=1)


def _rb_body(x_ref, m_ref):
    x = x_ref[...]
    m = jnp.max(x, axis=0, keepdims=True)
    s = jnp.sum(jnp.exp(x - m), axis=0, keepdims=True)
    m_ref[...] = jnp.concatenate([m, s], axis=0)


def _rc_body(nb, h0, h1, h2, h3, h4, x_ref, m_ref, bih_ref, bhh_ref,
             wg_ref, bg_ref, sg_ref, ro):
    i = pl.program_id(0)

    @pl.when(i == 0)
    def _():
        ro[...] = jnp.zeros_like(ro)

    m = m_ref[0:1]
    s = m_ref[1:2]
    w = jnp.exp(x_ref[...] - m) / s
    hrs = (h0, h1, h2, h3, h4)
    for t in range(5):
        wt = w[:, 3 + t:4 + t]
        ro[2 * t, :] += jnp.sum(wt * hrs[t][0], axis=0)
        ro[2 * t + 1, :] += jnp.sum(wt * hrs[t][1], axis=0)

    @pl.when(i == nb - 1)
    def _():
        q = _q_vec(bih_ref, bhh_ref)
        sg = jnp.zeros((1, 3), F32)
        for t in range(5):
            rows = (q[0:1], q[1:2], ro[2 * t:2 * t + 1],
                    ro[2 * t + 1:2 * t + 2])
            for j in range(4):
                sg = sg + jnp.dot(_bf(rows[j]), _bf(wg_ref[t, j]),
                                  preferred_element_type=F32)
            sg = sg + bg_ref[t:t + 1]
        sg_ref[...] = sg


def kernel(h, edge_index, e, W_emb, b_emb, Wmlp, bmlp, gamma, beta, Wn, bn_,
           Wg, bg, W_ih, W_hh, b_ih, b_hh):
    n = h.shape[0]
    hd = W_emb.shape[1]
    hh = hd // 2
    ll = Wmlp.shape[0]
    ne = edge_index.shape[1]
    r = 2000
    nb = n // r

    src = edge_index[0]
    dst = edge_index[1]
    src2 = jnp.concatenate([src, src + n])

    segsum = _make_segsum(n, hh, ne)

    full = lambda *shape: pl.BlockSpec(shape, lambda i: tuple(0 for _ in shape))
    rowblk3 = pl.BlockSpec((2, r, hh), lambda i: (0, i, 0))

    x = pl.pallas_call(
        _embed_body,
        grid=(nb,),
        in_specs=[pl.BlockSpec((r, 2), lambda i: (i, 0)),
                  full(2, hd), full(1, hd)],
        out_specs=rowblk3,
        out_shape=jax.ShapeDtypeStruct((2, n, hh), F32),
    )(h, W_emb, b_emb.reshape(1, hd))

    la = pl.pallas_call(
        functools.partial(_la_body, nb),
        grid=(nb,),
        in_specs=[rowblk3, rowblk3, full(hd, hd), full(2, hh)],
        out_specs=[rowblk3, full(4, hh)],
        out_shape=[jax.ShapeDtypeStruct((2, n, hh), F32),
                   jax.ShapeDtypeStruct((4, hh), F32)],
        scratch_shapes=[pltpu.VMEM((4, hh), F32)],
    )
    lb = pl.pallas_call(
        functools.partial(_lb_body, float(n)),
        grid=(nb,),
        in_specs=[rowblk3, rowblk3, full(4, hh), full(2, hh), full(2, hh)],
        out_specs=rowblk3,
        out_shape=jax.ShapeDtypeStruct((2, n, hh), F32),
    )

    hrs = [x]
    for l in range(ll):
        neigh = segsum(x.reshape(2 * n, hh), src2, dst).reshape(2, n, hh)
        t, st = la(x, neigh, Wmlp[l], bmlp[l].reshape(2, hh))
        x = lb(x, t, st, gamma[l].reshape(2, hh), beta[l].reshape(2, hh))
        hrs.append(x)

    nr = ll + 1
    X = pl.pallas_call(
        _ra_body,
        grid=(nb,),
        in_specs=[rowblk3] * nr + [full(nr, 2, hh, 3), full(nr, 3),
                                   full(8, hh), full(8, hh)],
        out_specs=pl.BlockSpec((r, 8), lambda i: (i, 0)),
        out_shape=jax.ShapeDtypeStruct((n, 8), F32),
    )(*hrs, Wn.reshape(nr, 2, hh, 3), bn_, b_ih.reshape(8, hh),
      b_hh.reshape(8, hh))

    M = pl.pallas_call(
        _rb_body,
        grid=(1,),
        in_specs=[full(n, 8)],
        out_specs=full(2, 8),
        out_shape=jax.ShapeDtypeStruct((2, 8), F32),
    )(X)

    sg = pl.pallas_call(
        functools.partial(_rc_body, nb),
        grid=(nb,),
        in_specs=[rowblk3] * nr + [pl.BlockSpec((r, 8), lambda i: (i, 0)),
                                   full(2, 8), full(8, hh), full(8, hh),
                                   full(nr, 4, hh, 3), full(nr, 3)],
        out_specs=full(1, 3),
        out_shape=jax.ShapeDtypeStruct((1, 3), F32),
        scratch_shapes=[pltpu.VMEM((16, hh), F32)],
    )(*hrs, X, M, b_ih.reshape(8, hh), b_hh.reshape(8, hh),
      Wg.reshape(nr, 4, hh, 3), bg)

    score_nodes = X[:, :3]
    return (score_nodes, sg)


# speedup vs baseline: 3.8060x; 1.2585x over previous
import functools

import jax
import jax.numpy as jnp
from jax import lax
from jax.experimental import pallas as pl
from jax.experimental.pallas import tpu as pltpu
from jax.experimental.pallas import tpu_sc as plsc

F32 = jnp.float32


def _bf(v):
    return v.astype(jnp.bfloat16)


def _dot(a, b):
    return jnp.dot(_bf(a), _bf(b), preferred_element_type=F32)


def _make_segsum(n, hh, e):
    ns = 16
    ept = e // ns
    ch = 40
    nch = ept // ch
    nring = 5
    unroll = 10
    own = (n // ns) & ~7
    tail = n - own * ns
    zsweep = -(-(own + tail) // ch)
    nfull = own // ch
    rem = own - nfull * ch
    assert ept % ch == 0 and e % ns == 0 and ch % 8 == 0
    assert rem % 8 == 0 and tail % 8 == 0 and own % 8 == 0
    assert nch % unroll == 0 and unroll % 2 == 0 and unroll % nring == 0

    mesh = plsc.VectorSubcoreMesh(core_axis_name="c", subcore_axis_name="s")

    @functools.partial(
        pl.kernel,
        mesh=mesh,
        out_type=jax.ShapeDtypeStruct((2 * n, hh), F32),
        scratch_types=(
            [pltpu.VMEM((ch, hh), F32)] * 2
            + [pltpu.VMEM((ch,), jnp.int32)] * (2 * nring)
            + [pltpu.VMEM_SHARED((n, hh), F32)]
            + [pltpu.SemaphoreType.DMA] * (4 + 2 * nring)
        ),
    )
    def segsum(x_hbm, src2_hbm, dst_hbm, out_hbm, *rest):
        rows = rest[:2]
        ibs = rest[2:2 + nring]
        ibd = rest[2 + nring:2 + 2 * nring]
        acc = rest[2 + 2 * nring]
        semg = rest[3 + 2 * nring:5 + 2 * nring]
        semsc = rest[5 + 2 * nring:7 + 2 * nring]
        semis = rest[7 + 2 * nring:7 + 3 * nring]
        semid = rest[7 + 3 * nring:]
        c = lax.axis_index("c")
        s = lax.axis_index("s")

        def zrow(i, carry):
            def zcol(j, carry2):
                rows[0][i, pl.ds(j * 16, 16)] = jnp.zeros((16,), F32)
                return carry2
            return lax.fori_loop(0, hh // 16, zcol, carry)
        lax.fori_loop(0, ch, zrow, 0)

        r0 = s * own
        for m in range(zsweep - 1):
            pltpu.sync_copy(rows[0], acc.at[pl.ds(r0 + m * ch, ch)])
        lastz = own + tail - (zsweep - 1) * ch
        pltpu.sync_copy(rows[0].at[pl.ds(0, lastz)],
                        acc.at[pl.ds(r0 + (zsweep - 1) * ch, lastz)])
        plsc.subcore_barrier()

        ebase_s = c * e + s * ept
        ebase_d = s * ept

        def idx_s(i, k):
            return pltpu.make_async_copy(
                src2_hbm.at[pl.ds(ebase_s + k * ch, ch)], ibs[i], semis[i])

        def idx_d(i, k):
            return pltpu.make_async_copy(
                dst_hbm.at[pl.ds(ebase_d + k * ch, ch)], ibd[i], semid[i])

        def gather(b, i):
            return pltpu.make_async_copy(x_hbm.at[ibs[i]], rows[b], semg[b])

        def scatter(b, i):
            return pltpu.make_async_copy(rows[b], acc.at[ibd[i]], semsc[b])

        for i in range(3):
            idx_s(i, i).start()
            idx_d(i, i).start()
        idx_s(0, 0).wait()
        idx_d(0, 0).wait()
        gather(0, 0).start()

        def outer(g, carry):
            for j in range(unroll):
                k = g * unroll + j
                b = j % 2
                i0 = j % nring
                i1 = (j + 1) % nring
                i3 = (j + 3) % nring
                gather(b, i0).wait()

                @pl.when(k + 3 < nch)
                def _():
                    idx_s(i3, k + 3).start()
                    idx_d(i3, k + 3).start()

                scatter(b, i0).start(add=True)

                @pl.when(k >= 1)
                def _():
                    scatter(1 - b, (j - 1) % nring).wait()

                @pl.when(k + 1 < nch)
                def _():
                    idx_s(i1, k + 1).wait()
                    idx_d(i1, k + 1).wait()
                    gather(1 - b, i1).start()
            return carry
        lax.fori_loop(0, nch // unroll, outer, 0)

        scatter((nch - 1) % 2, (nch - 1) % nring).wait()

        plsc.subcore_barrier()
        for m in range(nfull):
            pltpu.sync_copy(acc.at[pl.ds(r0 + m * ch, ch)],
                            out_hbm.at[pl.ds(c * n + r0 + m * ch, ch)])
        if rem:
            pltpu.sync_copy(acc.at[pl.ds(r0 + nfull * ch, rem)],
                            out_hbm.at[pl.ds(c * n + r0 + nfull * ch, rem)])

        @pl.when(s == ns - 1)
        def _():
            if tail:
                pltpu.sync_copy(acc.at[pl.ds(ns * own, tail)],
                                out_hbm.at[pl.ds(c * n + ns * own, tail)])

    return segsum


def _embed_body(h_ref, w_ref, b_ref, o_ref):
    hb = _bf(h_ref[...]).astype(F32)
    wb = _bf(w_ref[...]).astype(F32)
    x = hb[:, 0:1] * wb[0:1, :] + hb[:, 1:2] * wb[1:2, :] + b_ref[...]
    o_ref[0] = x[:, :128]
    o_ref[1] = x[:, 128:]


def _la_body(nb, x_ref, ng_ref, w_ref, b_ref, t_ref, st_ref, sums):
    i = pl.program_id(0)
    z0 = x_ref[0] + ng_ref[0]
    z1 = x_ref[1] + ng_ref[1]
    z0b, z1b = _bf(z0), _bf(z1)
    w = w_ref[...]
    t0 = (jnp.dot(z0b, _bf(w[:128, :128]), preferred_element_type=F32)
          + jnp.dot(z1b, _bf(w[128:, :128]), preferred_element_type=F32)
          + b_ref[0])
    t1 = (jnp.dot(z0b, _bf(w[:128, 128:]), preferred_element_type=F32)
          + jnp.dot(z1b, _bf(w[128:, 128:]), preferred_element_type=F32)
          + b_ref[1])
    t_ref[0] = t0
    t_ref[1] = t1

    @pl.when(i == 0)
    def _():
        sums[...] = jnp.zeros_like(sums)

    sums[0, :] += jnp.sum(t0, axis=0)
    sums[1, :] += jnp.sum(t1, axis=0)
    sums[2, :] += jnp.sum(t0 * t0, axis=0)
    sums[3, :] += jnp.sum(t1 * t1, axis=0)

    @pl.when(i == nb - 1)
    def _():
        st_ref[...] = sums[...]


def _lb_body(n, x_ref, t_ref, st_ref, g_ref, be_ref, o_ref):
    st = st_ref[...]
    mu0 = st[0:1] / n
    mu1 = st[1:2] / n
    var0 = st[2:3] / n - mu0 * mu0
    var1 = st[3:4] / n - mu1 * mu1
    t0 = (t_ref[0] - mu0) / jnp.sqrt(var0 + 1e-5) * g_ref[0:1] + be_ref[0:1]
    t1 = (t_ref[1] - mu1) / jnp.sqrt(var1 + 1e-5) * g_ref[1:2] + be_ref[1:2]
    o_ref[0] = x_ref[0] + jnp.maximum(t0, 0.0)
    o_ref[1] = x_ref[1] + jnp.maximum(t1, 0.0)


def _q_vec(bih_ref, bhh_ref):
    g = bih_ref[...] + bhh_ref[...]
    i_g, f_g, g_g, o_g = g[0:2], g[2:4], g[4:6], g[6:8]
    del f_g
    c = jax.nn.sigmoid(i_g) * jnp.tanh(g_g)
    return jax.nn.sigmoid(o_g) * jnp.tanh(c)


def _ra_body(h0, h1, h2, h3, h4, wn_ref, bn_ref, bih_ref, bhh_ref, x_ref):
    q = _q_vec(bih_ref, bhh_ref)
    hrs = (h0, h1, h2, h3, h4)
    sn = jnp.sum(bn_ref[...], axis=0, keepdims=True)
    sn = jnp.broadcast_to(sn, (h0.shape[1], 3))
    atts = []
    for t in range(5):
        hr = hrs[t]
        sn = (sn + jnp.dot(_bf(hr[0]), _bf(wn_ref[t, 0]),
                           preferred_element_type=F32)
              + jnp.dot(_bf(hr[1]), _bf(wn_ref[t, 1]),
                        preferred_element_type=F32))
        att = (jnp.sum(hr[0] * q[0:1], axis=1, keepdims=True)
               + jnp.sum(hr[1] * q[1:2], axis=1, keepdims=True))
        atts.append(att)
    x_ref[...] = jnp.concatenate([sn] + atts, axis=1)


def _rb_body(x_ref, m_ref):
    x = x_ref[...]
    m = jnp.max(x, axis=0, keepdims=True)
    s = jnp.sum(jnp.exp(x - m), axis=0, keepdims=True)
    m_ref[...] = jnp.concatenate([m, s], axis=0)


def _rc_body(nb, h0, h1, h2, h3, h4, x_ref, m_ref, bih_ref, bhh_ref,
             wg_ref, bg_ref, sg_ref, ro):
    i = pl.program_id(0)

    @pl.when(i == 0)
    def _():
        ro[...] = jnp.zeros_like(ro)

    m = m_ref[0:1]
    s = m_ref[1:2]
    w = jnp.exp(x_ref[...] - m) / s
    hrs = (h0, h1, h2, h3, h4)
    for t in range(5):
        wt = w[:, 3 + t:4 + t]
        ro[2 * t, :] += jnp.sum(wt * hrs[t][0], axis=0)
        ro[2 * t + 1, :] += jnp.sum(wt * hrs[t][1], axis=0)

    @pl.when(i == nb - 1)
    def _():
        q = _q_vec(bih_ref, bhh_ref)
        sg = jnp.zeros((1, 3), F32)
        for t in range(5):
            rows = (q[0:1], q[1:2], ro[2 * t:2 * t + 1],
                    ro[2 * t + 1:2 * t + 2])
            for j in range(4):
                sg = sg + jnp.dot(_bf(rows[j]), _bf(wg_ref[t, j]),
                                  preferred_element_type=F32)
            sg = sg + bg_ref[t:t + 1]
        sg_ref[...] = sg


def kernel(h, edge_index, e, W_emb, b_emb, Wmlp, bmlp, gamma, beta, Wn, bn_,
           Wg, bg, W_ih, W_hh, b_ih, b_hh):
    n = h.shape[0]
    hd = W_emb.shape[1]
    hh = hd // 2
    ll = Wmlp.shape[0]
    ne = edge_index.shape[1]
    r = 2000
    nb = n // r

    src = edge_index[0]
    dst = edge_index[1]
    src2 = jnp.concatenate([src, src + n])

    segsum = _make_segsum(n, hh, ne)

    full = lambda *shape: pl.BlockSpec(shape, lambda i: tuple(0 for _ in shape))
    rowblk3 = pl.BlockSpec((2, r, hh), lambda i: (0, i, 0))

    x = pl.pallas_call(
        _embed_body,
        grid=(nb,),
        in_specs=[pl.BlockSpec((r, 2), lambda i: (i, 0)),
                  full(2, hd), full(1, hd)],
        out_specs=rowblk3,
        out_shape=jax.ShapeDtypeStruct((2, n, hh), F32),
    )(h, W_emb, b_emb.reshape(1, hd))

    la = pl.pallas_call(
        functools.partial(_la_body, nb),
        grid=(nb,),
        in_specs=[rowblk3, rowblk3, full(hd, hd), full(2, hh)],
        out_specs=[rowblk3, full(4, hh)],
        out_shape=[jax.ShapeDtypeStruct((2, n, hh), F32),
                   jax.ShapeDtypeStruct((4, hh), F32)],
        scratch_shapes=[pltpu.VMEM((4, hh), F32)],
    )
    lb = pl.pallas_call(
        functools.partial(_lb_body, float(n)),
        grid=(nb,),
        in_specs=[rowblk3, rowblk3, full(4, hh), full(2, hh), full(2, hh)],
        out_specs=rowblk3,
        out_shape=jax.ShapeDtypeStruct((2, n, hh), F32),
    )

    hrs = [x]
    for l in range(ll):
        neigh = segsum(x.reshape(2 * n, hh), src2, dst).reshape(2, n, hh)
        t, st = la(x, neigh, Wmlp[l], bmlp[l].reshape(2, hh))
        x = lb(x, t, st, gamma[l].reshape(2, hh), beta[l].reshape(2, hh))
        hrs.append(x)

    nr = ll + 1
    X = pl.pallas_call(
        _ra_body,
        grid=(nb,),
        in_specs=[rowblk3] * nr + [full(nr, 2, hh, 3), full(nr, 3),
                                   full(8, hh), full(8, hh)],
        out_specs=pl.BlockSpec((r, 8), lambda i: (i, 0)),
        out_shape=jax.ShapeDtypeStruct((n, 8), F32),
    )(*hrs, Wn.reshape(nr, 2, hh, 3), bn_, b_ih.reshape(8, hh),
      b_hh.reshape(8, hh))

    M = pl.pallas_call(
        _rb_body,
        grid=(1,),
        in_specs=[full(n, 8)],
        out_specs=full(2, 8),
        out_shape=jax.ShapeDtypeStruct((2, 8), F32),
    )(X)

    sg = pl.pallas_call(
        functools.partial(_rc_body, nb),
        grid=(nb,),
        in_specs=[rowblk3] * nr + [pl.BlockSpec((r, 8), lambda i: (i, 0)),
                                   full(2, 8), full(8, hh), full(8, hh),
                                   full(nr, 4, hh, 3), full(nr, 3)],
        out_specs=full(1, 3),
        out_shape=jax.ShapeDtypeStruct((1, 3), F32),
        scratch_shapes=[pltpu.VMEM((16, hh), F32)],
    )(*hrs, X, M, b_ih.reshape(8, hh), b_hh.reshape(8, hh),
      Wg.reshape(nr, 4, hh, 3), bg)

    score_nodes = X[:, :3]
    return (score_nodes, sg)
